# baseline (speedup 1.0000x reference)
import jax
import jax.numpy as jnp
from jax.experimental import pallas as pl

_NH = 16
_EPS = 1e-10
_L = 128


def _select_body(lon_ref, lat_ref, dist_out, idx_out, lon_out, lat_out):
    lon = lon_ref[...]
    lat = lat_ref[...]
    r, s = lon.shape
    nc = s // _L
    dist = jnp.sqrt(lon * lon + lat * lat + 1e-12)
    lon_c = [lon[:, c * _L:(c + 1) * _L] for c in range(nc)]
    lat_c = [lat[:, c * _L:(c + 1) * _L] for c in range(nc)]
    lane_iota = jax.lax.broadcasted_iota(jnp.int32, (r, _L), 1)
    big_i = jnp.int32(2 ** 30)

    def lane_min(dist):
        chunks = [dist[:, c * _L:(c + 1) * _L] for c in range(nc)]
        lm = chunks[0]
        for c in range(1, nc):
            lm = jnp.minimum(lm, chunks[c])
        return lm

    def merge(bv, bi, blon, blat, cv, ci, clon, clat):
        vv = jnp.concatenate([bv, cv], axis=1)
        ii = jnp.concatenate([bi, ci], axis=1)
        ll = jnp.concatenate([blon, clon], axis=1)
        tt = jnp.concatenate([blat, clat], axis=1)
        nbv, nbi, nbl, nbt = [], [], [], []
        for _ in range(_NH):
            m = jnp.min(vv, axis=1, keepdims=True)
            am = jnp.min(jnp.where(vv == m, ii, big_i), axis=1, keepdims=True)
            selm = ii == am
            nbv.append(m)
            nbi.append(am)
            nbl.append(jnp.sum(jnp.where(selm, ll, 0.0), axis=1, keepdims=True))
            nbt.append(jnp.sum(jnp.where(selm, tt, 0.0), axis=1, keepdims=True))
            vv = jnp.where(selm, jnp.inf, vv)
        return (jnp.concatenate(nbv, 1), jnp.concatenate(nbi, 1),
                jnp.concatenate(nbl, 1), jnp.concatenate(nbt, 1))

    def body(carry):
        dist, lm, bv, bi, blon, blat, _ = carry
        chunks = [dist[:, c * _L:(c + 1) * _L] for c in range(nc)]
        pos = jnp.full((r, _L), nc, jnp.int32)
        clon = jnp.zeros((r, _L), jnp.float32)
        clat = jnp.zeros((r, _L), jnp.float32)
        for c in range(nc - 1, -1, -1):
            hit = chunks[c] == lm
            pos = jnp.where(hit, c, pos)
            clon = jnp.where(hit, lon_c[c], clon)
            clat = jnp.where(hit, lat_c[c], clat)
        ci = pos * _L + lane_iota
        new_chunks = [jnp.where(pos == c, jnp.inf, chunks[c]) for c in range(nc)]
        dist = jnp.concatenate(new_chunks, axis=1)
        bv, bi, blon, blat = merge(bv, bi, blon, blat, lm, ci, clon, clat)
        lm = lane_min(dist)
        gmin = jnp.min(lm, axis=1)
        done = jnp.all(bv[:, _NH - 1] < gmin)
        return dist, lm, bv, bi, blon, blat, done

    init = (dist, lane_min(dist),
            jnp.full((r, _NH), jnp.inf, jnp.float32),
            jnp.zeros((r, _NH), jnp.int32),
            jnp.zeros((r, _NH), jnp.float32),
            jnp.zeros((r, _NH), jnp.float32),
            jnp.bool_(False))
    out = jax.lax.while_loop(lambda c: jnp.logical_not(c[-1]), body, init)
    _, _, bv, bi, blon, blat, _ = out
    dist_out[...] = bv
    idx_out[...] = bi
    lon_out[...] = blon
    lat_out[...] = blat


def kernel(x, d_lon, d_lat):
    t, s = d_lon.shape
    r = 8
    grid = t // r
    out_shapes = (
        jax.ShapeDtypeStruct((t, _NH), jnp.float32),
        jax.ShapeDtypeStruct((t, _NH), jnp.int32),
        jax.ShapeDtypeStruct((t, _NH), jnp.float32),
        jax.ShapeDtypeStruct((t, _NH), jnp.float32),
    )
    in_spec = pl.BlockSpec((r, s), lambda i: (i, 0))
    out_spec = pl.BlockSpec((r, _NH), lambda i: (i, 0))
    dist_sel, idx, lon_sel, lat_sel = pl.pallas_call(
        _select_body,
        grid=(grid,),
        in_specs=[in_spec, in_spec],
        out_specs=[out_spec, out_spec, out_spec, out_spec],
        out_shape=out_shapes,
    )(d_lon, d_lat)

    x_nearest = jnp.take(x, idx, axis=1)
    w = 1.0 / (dist_sel + _EPS)
    w = w / jnp.sum(w, axis=-1, keepdims=True)
    x_inter = jnp.sum(x_nearest * w[None, :, :], axis=-1)
    return (x_nearest, x_inter, dist_sel, lon_sel, lat_sel)


# speedup vs baseline: 2.6433x; 2.6433x over previous
import jax
import jax.numpy as jnp
from jax.experimental import pallas as pl
from jax.experimental.pallas import tpu as pltpu

_NH = 16
_EPS = 1e-10
_L = 128


def _tree(op, xs):
    xs = list(xs)
    while len(xs) > 1:
        nxt = [op(xs[i], xs[i + 1]) for i in range(0, len(xs) - 1, 2)]
        if len(xs) % 2:
            nxt.append(xs[-1])
        xs = nxt
    return xs[0]


def _select_body(lon_ref, lat_ref, dist_out, idx_out, dist_ref):
    lon = lon_ref[...]
    lat = lat_ref[...]
    r, s = lon.shape
    nc = s // _L
    dist_ref[...] = jnp.sqrt(lon * lon + lat * lat + 1e-12)
    lane_iota = jax.lax.broadcasted_iota(jnp.int32, (r, _L), 1)
    big_i = jnp.int32(2 ** 30)

    def chunk(c):
        return dist_ref[:, c * _L:(c + 1) * _L]

    def lane_min():
        return _tree(jnp.minimum, [chunk(c) for c in range(nc)])

    def merge(bv, bi, cv, ci):
        vv = jnp.concatenate([bv, cv], axis=1)
        ii = jnp.concatenate([bi, ci], axis=1)
        nbv, nbi = [], []
        for _ in range(_NH):
            m = jnp.min(vv, axis=1, keepdims=True)
            am = jnp.min(jnp.where(vv == m, ii, big_i), axis=1, keepdims=True)
            nbv.append(m)
            nbi.append(am)
            vv = jnp.where(ii == am, jnp.inf, vv)
        return jnp.concatenate(nbv, 1), jnp.concatenate(nbi, 1)

    def body(carry):
        _, lm, bv, bi = carry
        chunks = [chunk(c) for c in range(nc)]
        pos = _tree(jnp.minimum,
                    [jnp.where(chunks[c] == lm, c, big_i) for c in range(nc)])
        new_chunks = [jnp.where(pos == c, jnp.inf, chunks[c]) for c in range(nc)]
        dist_ref[...] = jnp.concatenate(new_chunks, axis=1)
        bv, bi = merge(bv, bi, lm, pos * _L + lane_iota)
        lm = _tree(jnp.minimum, new_chunks)
        gmin = jnp.min(lm, axis=1)
        done = jnp.all(bv[:, _NH - 1] < gmin)
        return done, lm, bv, bi

    init = (jnp.bool_(False), lane_min(),
            jnp.full((r, _NH), jnp.inf, jnp.float32),
            jnp.zeros((r, _NH), jnp.int32))
    _, _, bv, bi = jax.lax.while_loop(lambda c: jnp.logical_not(c[0]),
                                      body, init)
    dist_out[...] = bv
    idx_out[...] = bi


def kernel(x, d_lon, d_lat):
    t, s = d_lon.shape
    r = min(32, t)
    grid = t // r
    out_shapes = (
        jax.ShapeDtypeStruct((t, _NH), jnp.float32),
        jax.ShapeDtypeStruct((t, _NH), jnp.int32),
    )
    in_spec = pl.BlockSpec((r, s), lambda i: (i, 0))
    out_spec = pl.BlockSpec((r, _NH), lambda i: (i, 0))
    dist_sel, idx = pl.pallas_call(
        _select_body,
        grid=(grid,),
        in_specs=[in_spec, in_spec],
        out_specs=[out_spec, out_spec],
        out_shape=out_shapes,
        scratch_shapes=[pltpu.VMEM((r, s), jnp.float32)],
    )(d_lon, d_lat)

    lon_sel = jnp.take_along_axis(d_lon, idx, axis=1)
    lat_sel = jnp.take_along_axis(d_lat, idx, axis=1)
    x_nearest = jnp.take(x, idx, axis=1)
    w = 1.0 / (dist_sel + _EPS)
    w = w / jnp.sum(w, axis=-1, keepdims=True)
    x_inter = jnp.sum(x_nearest * w[None, :, :], axis=-1)
    return (x_nearest, x_inter, dist_sel, lon_sel, lat_sel)
